# Initial kernel scaffold; baseline (speedup 1.0000x reference)
import jax
import jax.numpy as jnp
from jax.experimental import pallas as pl


def kernel(features, labels, centers):
    raise NotImplementedError("write your pallas kernel here")


# speedup vs baseline: 2.6061x; 2.6061x over previous
import functools

import jax
import jax.numpy as jnp
from jax import lax
from jax.experimental import pallas as pl
from jax.experimental.pallas import tpu as pltpu
from jax.experimental.pallas import tpu_sc as plsc

NUM_CLASSES = 100000
FEAT_DIM = 128
BATCH = 16384

NC = 2
NS = 16
NW = NC * NS
BPW = BATCH // NW
CHUNK = 128
NCHUNK = BPW // CHUNK

_sc_mesh = plsc.VectorSubcoreMesh(core_axis_name="c", subcore_axis_name="s")


@functools.partial(
    pl.kernel,
    mesh=_sc_mesh,
    out_type=jax.ShapeDtypeStruct((BATCH, FEAT_DIM), jnp.float32),
    scratch_types=[
        pltpu.VMEM((BPW,), jnp.int32),
        pltpu.VMEM((BPW, FEAT_DIM), jnp.float32),
        pltpu.SemaphoreType.DMA,
    ],
)
def _sc_gather(centers_hbm, idx_hbm, out_hbm, idx_v, rows_v, sem):
    wid = lax.axis_index("s") * NC + lax.axis_index("c")
    base = wid * BPW
    pltpu.sync_copy(idx_hbm.at[pl.ds(base, BPW)], idx_v)
    handles = []
    for j in range(NCHUNK):
        handles.append(
            pltpu.async_copy(
                centers_hbm.at[idx_v.at[pl.ds(j * CHUNK, CHUNK)]],
                rows_v.at[pl.ds(j * CHUNK, CHUNK)],
                sem,
            )
        )
    for h in handles:
        h.wait()
    pltpu.sync_copy(rows_v, out_hbm.at[pl.ds(base, BPW)])


_TC_BLK = 2048


def _tc_loss_body(f_ref, c_ref, out_ref, acc_ref):
    i = pl.program_id(0)
    n = pl.num_programs(0)
    f = f_ref[...]
    c = c_ref[...]
    sf = jnp.sum(f * f, axis=1, keepdims=True)
    sc = jnp.sum(c * c, axis=1, keepdims=True)
    fc = jnp.sum(f * c, axis=1, keepdims=True)
    eps = jnp.float32(1e-12)
    denom = jnp.maximum(jnp.sqrt(sf), eps) * jnp.maximum(jnp.sqrt(sc), eps)
    part = jnp.sum(fc / denom)

    @pl.when(i == 0)
    def _():
        acc_ref[0] = 0.0

    acc_ref[0] += part

    @pl.when(i == n - 1)
    def _():
        out_ref[0, 0] = 1.0 - acc_ref[0] / jnp.float32(BATCH)


_tc_loss = pl.pallas_call(
    _tc_loss_body,
    grid=(BATCH // _TC_BLK,),
    in_specs=[
        pl.BlockSpec((_TC_BLK, FEAT_DIM), lambda i: (i, 0)),
        pl.BlockSpec((_TC_BLK, FEAT_DIM), lambda i: (i, 0)),
    ],
    out_specs=pl.BlockSpec((1, 1), lambda i: (0, 0), memory_space=pltpu.SMEM),
    out_shape=jax.ShapeDtypeStruct((1, 1), jnp.float32),
    scratch_shapes=[pltpu.SMEM((1,), jnp.float32)],
)


def kernel(features, labels, centers):
    idx = labels.astype(jnp.int32)
    gathered = _sc_gather(centers, idx)
    loss = _tc_loss(features, gathered)
    return loss[0, 0]
